# baseline (speedup 1.0000x reference)
import functools

import jax
import jax.numpy as jnp
from jax.experimental import pallas as pl
from jax.experimental.pallas import tpu as pltpu

G = 4
DIM = 32
NE = 8
OC = 64
K = 4
LL = 128
LP = 126
TB = 32


def _moe_body(x_ref, gin_ref, w1_ref, b1_ref, w2_ref, b2_ref, wg_ref,
              cf_ref, gates_ref, loss_ref, imp_ref, load_ref, *, nbt):
    g = pl.program_id(0)
    bt = pl.program_id(1)

    gin = gin_ref[0]
    wg = wg_ref[0]
    logits = jax.lax.dot_general(gin, wg, (((1,), (0,)), ((), ())),
                                 preferred_element_type=jnp.float32)
    logits = logits - jnp.max(logits, axis=1, keepdims=True)
    ex = jnp.exp(logits)
    probs = ex / jnp.sum(ex, axis=1, keepdims=True)

    iota = jax.lax.broadcasted_iota(jnp.int32, (TB, NE), 1)
    vals = probs
    sel = jnp.zeros((TB, NE), jnp.bool_)
    for _ in range(K):
        m = jnp.max(vals, axis=1, keepdims=True)
        cand = jnp.where(vals == m, iota, NE)
        first = jnp.min(cand, axis=1, keepdims=True)
        pick = iota == first
        sel = jnp.logical_or(sel, pick)
        vals = jnp.where(pick, -jnp.inf, vals)
    tk = jnp.where(sel, probs, 0.0)
    denom = jnp.sum(tk, axis=1, keepdims=True) + 1e-6
    gates = tk / denom
    gates_ref[0] = gates

    imp_part = jnp.sum(gates, axis=0, keepdims=True)
    load_part = jnp.sum((gates > 0).astype(jnp.float32), axis=0,
                        keepdims=True)

    @pl.when(bt == 0)
    def _():
        imp_ref[pl.ds(g, 1), :] = imp_part
        load_ref[pl.ds(g, 1), :] = load_part

    @pl.when(bt != 0)
    def _():
        imp_ref[pl.ds(g, 1), :] = imp_ref[pl.ds(g, 1), :] + imp_part
        load_ref[pl.ds(g, 1), :] = load_ref[pl.ds(g, 1), :] + load_part

    xb = x_ref[0]
    c2 = jnp.concatenate(
        [xb[:, 0:LP, :], xb[:, 1:LP + 1, :], xb[:, 2:LP + 2, :]], axis=2)
    w1 = w1_ref[0]
    hpre = jax.lax.dot_general(c2, w1, (((2,), (0,)), ((), ())),
                               preferred_element_type=jnp.float32)
    h = jnp.tanh(hpre + b1_ref[0][None])

    w2 = w2_ref[0]
    raw = jax.lax.dot_general(h, w2, (((2,), (0,)), ((), ())),
                              preferred_element_type=jnp.float32)
    bias_c = jax.lax.dot_general(gates, b2_ref[0], (((1,), (0,)), ((), ())),
                                 preferred_element_type=jnp.float32)
    out = jnp.broadcast_to(bias_c[:, None, :], (TB, LP, OC))
    for e in range(NE):
        ge = gates[:, e][:, None, None]
        out = out + ge * raw[:, :, OC * e:OC * (e + 1)]
    cf_ref[:, 0] = jnp.swapaxes(out, 1, 2)

    @pl.when(jnp.logical_and(g == G - 1, bt == nbt - 1))
    def _():
        def cv(v):
            m = jnp.mean(v, axis=1, keepdims=True)
            var = jnp.sum((v - m) ** 2, axis=1, keepdims=True) / (NE - 1)
            return var / (m * m + 1e-10)
        loss = 0.01 * (jnp.sum(cv(imp_ref[...]), keepdims=True)
                       + jnp.sum(cv(load_ref[...]), keepdims=True))
        loss_ref[...] = loss


def _moe_call(xT4, gin, w1, b1, w2p, b2r, wg, interpret=False):
    b = xT4.shape[1]
    nbt = b // TB
    grid = (G, nbt)
    out_shapes = (
        jax.ShapeDtypeStruct((b, G, OC, LP), jnp.float32),
        jax.ShapeDtypeStruct((G, b, NE), jnp.float32),
        jax.ShapeDtypeStruct((1, 1), jnp.float32),
    )
    return pl.pallas_call(
        functools.partial(_moe_body, nbt=nbt),
        grid=grid,
        in_specs=[
            pl.BlockSpec((1, TB, LL, DIM), lambda g, t: (g, t, 0, 0)),
            pl.BlockSpec((1, TB, DIM * 5), lambda g, t: (g, t, 0)),
            pl.BlockSpec((1, 3 * DIM, OC), lambda g, t: (g, 0, 0)),
            pl.BlockSpec((1, 1, OC), lambda g, t: (g, 0, 0)),
            pl.BlockSpec((1, OC, OC * NE), lambda g, t: (g, 0, 0)),
            pl.BlockSpec((1, NE, OC), lambda g, t: (g, 0, 0)),
            pl.BlockSpec((1, DIM * 5, NE), lambda g, t: (g, 0, 0)),
        ],
        out_specs=(
            pl.BlockSpec((TB, 1, OC, LP), lambda g, t: (t, g, 0, 0)),
            pl.BlockSpec((1, TB, NE), lambda g, t: (g, t, 0)),
            pl.BlockSpec((1, 1), lambda g, t: (0, 0)),
        ),
        out_shape=out_shapes,
        scratch_shapes=[
            pltpu.VMEM((G, NE), jnp.float32),
            pltpu.VMEM((G, NE), jnp.float32),
        ],
        interpret=interpret,
    )(xT4, gin, w1, b1, w2p, b2r, wg)


def _run(x, conv1_w, conv1_b, conv2_w, conv2_b, w_gate, interpret=False):
    b = x.shape[0]
    xT4 = x.transpose(0, 2, 1).reshape(b, LL, G, DIM).transpose(2, 0, 1, 3)
    gin = x[:, :, LL - 6:LL - 1].reshape(b, G, DIM * 5).transpose(1, 0, 2)
    w1 = conv1_w.transpose(0, 3, 2, 1).reshape(G, 3 * DIM, OC)
    b1 = conv1_b.reshape(G, 1, OC)
    w2p = conv2_w[..., 0].reshape(G, OC, NE, OC).transpose(0, 3, 2, 1) \
        .reshape(G, OC, OC * NE)
    b2r = conv2_b.reshape(G, OC, NE).transpose(0, 2, 1)
    cf, gates_all, loss2d = _moe_call(xT4, gin, w1, b1, w2p, b2r,
                                      w_gate, interpret=interpret)
    return cf, loss2d[0, 0], gates_all.transpose(1, 2, 0)


def kernel(x, conv1_w, conv1_b, conv2_w, conv2_b, w_gate):
    return _run(x, conv1_w, conv1_b, conv2_w, conv2_b, w_gate)


# speedup vs baseline: 1.4821x; 1.4821x over previous
import functools

import jax
import jax.numpy as jnp
from jax.experimental import pallas as pl
from jax.experimental.pallas import tpu as pltpu

G = 4
DIM = 32
NE = 8
OC = 64
K = 4
LL = 128
LP = 126
TB = 32


def _moe_body(x_ref, gin_ref, w1_ref, b1_ref, w2_ref, b2_ref, wg_ref, r_ref,
              cf_ref, gates_ref, loss_ref, imp_ref, load_ref, *, nbt):
    g = pl.program_id(0)
    bt = pl.program_id(1)

    gin = gin_ref[0]
    wg = wg_ref[0]
    logits = jax.lax.dot_general(gin, wg, (((1,), (0,)), ((), ())),
                                 preferred_element_type=jnp.float32)
    logits = logits - jnp.max(logits, axis=1, keepdims=True)
    ex = jnp.exp(logits)
    probs = ex / jnp.sum(ex, axis=1, keepdims=True)

    iota = jax.lax.broadcasted_iota(jnp.int32, (TB, NE), 1)
    vals = probs
    sel = jnp.zeros((TB, NE), jnp.bool_)
    for _ in range(K):
        m = jnp.max(vals, axis=1, keepdims=True)
        cand = jnp.where(vals == m, iota, NE)
        first = jnp.min(cand, axis=1, keepdims=True)
        pick = iota == first
        sel = jnp.logical_or(sel, pick)
        vals = jnp.where(pick, -jnp.inf, vals)
    tk = jnp.where(sel, probs, 0.0)
    denom = jnp.sum(tk, axis=1, keepdims=True) + 1e-6
    gates = tk / denom
    gates_ref[0] = gates

    imp_part = jnp.sum(gates, axis=0, keepdims=True)
    load_part = jnp.sum((gates > 0).astype(jnp.float32), axis=0,
                        keepdims=True)

    @pl.when(bt == 0)
    def _():
        imp_ref[pl.ds(g, 1), :] = imp_part
        load_ref[pl.ds(g, 1), :] = load_part

    @pl.when(bt != 0)
    def _():
        imp_ref[pl.ds(g, 1), :] = imp_ref[pl.ds(g, 1), :] + imp_part
        load_ref[pl.ds(g, 1), :] = load_ref[pl.ds(g, 1), :] + load_part

    xb = x_ref[0]
    c2 = jnp.concatenate(
        [xb[:, 0:LP, :], xb[:, 1:LP + 1, :], xb[:, 2:LP + 2, :]], axis=2)
    w1 = w1_ref[0]
    hpre = jax.lax.dot_general(c2, w1, (((2,), (0,)), ((), ())),
                               preferred_element_type=jnp.float32)
    h = jnp.tanh(hpre + b1_ref[0][None])

    w2 = w2_ref[0]
    raw = jax.lax.dot_general(h, w2, (((2,), (0,)), ((), ())),
                              preferred_element_type=jnp.float32)
    bias_c = jax.lax.dot_general(gates, b2_ref[0], (((1,), (0,)), ((), ())),
                                 preferred_element_type=jnp.float32)
    ge2 = jax.lax.dot_general(gates, r_ref[0], (((1,), (0,)), ((), ())),
                              preferred_element_type=jnp.float32)
    prod = raw * ge2[:, None, :]
    out = bias_c[:, None, :]
    for e in range(NE):
        out = out + prod[:, :, OC * e:OC * (e + 1)]
    cf_ref[:, 0] = jnp.swapaxes(out, 1, 2)

    @pl.when(jnp.logical_and(g == G - 1, bt == nbt - 1))
    def _():
        def cv(v):
            m = jnp.mean(v, axis=1, keepdims=True)
            var = jnp.sum((v - m) ** 2, axis=1, keepdims=True) / (NE - 1)
            return var / (m * m + 1e-10)
        loss = 0.01 * (jnp.sum(cv(imp_ref[...]), keepdims=True)
                       + jnp.sum(cv(load_ref[...]), keepdims=True))
        loss_ref[...] = loss


def _moe_call(xT4, gin, w1, b1, w2p, b2r, wg, rsel, interpret=False):
    b = xT4.shape[1]
    nbt = b // TB
    grid = (G, nbt)
    out_shapes = (
        jax.ShapeDtypeStruct((b, G, OC, LP), jnp.float32),
        jax.ShapeDtypeStruct((G, b, NE), jnp.float32),
        jax.ShapeDtypeStruct((1, 1), jnp.float32),
    )
    return pl.pallas_call(
        functools.partial(_moe_body, nbt=nbt),
        grid=grid,
        in_specs=[
            pl.BlockSpec((1, TB, LL, DIM), lambda g, t: (g, t, 0, 0)),
            pl.BlockSpec((1, TB, DIM * 5), lambda g, t: (g, t, 0)),
            pl.BlockSpec((1, 3 * DIM, OC), lambda g, t: (g, 0, 0)),
            pl.BlockSpec((1, 1, OC), lambda g, t: (g, 0, 0)),
            pl.BlockSpec((1, OC, OC * NE), lambda g, t: (g, 0, 0)),
            pl.BlockSpec((1, NE, OC), lambda g, t: (g, 0, 0)),
            pl.BlockSpec((1, DIM * 5, NE), lambda g, t: (g, 0, 0)),
            pl.BlockSpec((1, NE, OC * NE), lambda g, t: (0, 0, 0)),
        ],
        out_specs=(
            pl.BlockSpec((TB, 1, OC, LP), lambda g, t: (t, g, 0, 0)),
            pl.BlockSpec((1, TB, NE), lambda g, t: (g, t, 0)),
            pl.BlockSpec((1, 1), lambda g, t: (0, 0)),
        ),
        out_shape=out_shapes,
        scratch_shapes=[
            pltpu.VMEM((G, NE), jnp.float32),
            pltpu.VMEM((G, NE), jnp.float32),
        ],
        interpret=interpret,
    )(xT4, gin, w1, b1, w2p, b2r, wg, rsel)


def _run(x, conv1_w, conv1_b, conv2_w, conv2_b, w_gate, interpret=False):
    b = x.shape[0]
    xT4 = x.transpose(0, 2, 1).reshape(b, LL, G, DIM).transpose(2, 0, 1, 3)
    gin = x[:, :, LL - 6:LL - 1].reshape(b, G, DIM * 5).transpose(1, 0, 2)
    w1 = conv1_w.transpose(0, 3, 2, 1).reshape(G, 3 * DIM, OC)
    b1 = conv1_b.reshape(G, 1, OC)
    w2p = conv2_w[..., 0].reshape(G, OC, NE, OC).transpose(0, 3, 2, 1) \
        .reshape(G, OC, OC * NE)
    b2r = conv2_b.reshape(G, OC, NE).transpose(0, 2, 1)
    eye = jnp.eye(NE, dtype=jnp.float32)
    rsel = jnp.repeat(eye, OC, axis=1).reshape(1, NE, OC * NE)
    cf, gates_all, loss2d = _moe_call(xT4, gin, w1, b1, w2p, b2r,
                                      w_gate, rsel, interpret=interpret)
    return cf, loss2d[0, 0], gates_all.transpose(1, 2, 0)


def kernel(x, conv1_w, conv1_b, conv2_w, conv2_b, w_gate):
    return _run(x, conv1_w, conv1_b, conv2_w, conv2_b, w_gate)


# speedup vs baseline: 2.3400x; 1.5788x over previous
import functools

import jax
import jax.numpy as jnp
from jax.experimental import pallas as pl
from jax.experimental.pallas import tpu as pltpu

G = 4
DIM = 32
NE = 8
OC = 64
K = 4
LL = 128
LP = 126
TB = 32


def _moe_body(x_ref, gin_ref, w1_ref, b1_ref, w2_ref, b2_ref, wg_ref,
              cf_ref, gates_ref, loss_ref, imp_ref, load_ref, *, nbt):
    g = pl.program_id(0)
    bt = pl.program_id(1)

    gin = gin_ref[0]
    wg = wg_ref[0]
    logits = jax.lax.dot_general(gin, wg, (((1,), (0,)), ((), ())),
                                 preferred_element_type=jnp.float32)
    logits = logits - jnp.max(logits, axis=1, keepdims=True)
    ex = jnp.exp(logits)
    probs = ex / jnp.sum(ex, axis=1, keepdims=True)

    iota = jax.lax.broadcasted_iota(jnp.int32, (TB, NE), 1)
    vals = probs
    sel = jnp.zeros((TB, NE), jnp.bool_)
    for _ in range(K):
        m = jnp.max(vals, axis=1, keepdims=True)
        cand = jnp.where(vals == m, iota, NE)
        first = jnp.min(cand, axis=1, keepdims=True)
        pick = iota == first
        sel = jnp.logical_or(sel, pick)
        vals = jnp.where(pick, -jnp.inf, vals)
    tk = jnp.where(sel, probs, 0.0)
    denom = jnp.sum(tk, axis=1, keepdims=True) + 1e-6
    gates = tk / denom
    gates_ref[0] = gates

    imp_part = jnp.sum(gates, axis=0, keepdims=True)
    load_part = jnp.sum((gates > 0).astype(jnp.float32), axis=0,
                        keepdims=True)

    @pl.when(bt == 0)
    def _():
        imp_ref[pl.ds(g, 1), :] = imp_part
        load_ref[pl.ds(g, 1), :] = load_part

    @pl.when(bt != 0)
    def _():
        imp_ref[pl.ds(g, 1), :] = imp_ref[pl.ds(g, 1), :] + imp_part
        load_ref[pl.ds(g, 1), :] = load_ref[pl.ds(g, 1), :] + load_part

    xb = x_ref[0]
    c2 = jnp.concatenate(
        [xb[:, 0:LP, :], xb[:, 1:LP + 1, :], xb[:, 2:LP + 2, :]], axis=2)
    w1 = w1_ref[0]
    hpre = jax.lax.dot_general(c2, w1, (((2,), (0,)), ((), ())),
                               preferred_element_type=jnp.float32)
    h = jnp.tanh(hpre + b1_ref[0][None])

    bias_c = jax.lax.dot_general(gates, b2_ref[0], (((1,), (0,)), ((), ())),
                                 preferred_element_type=jnp.float32)
    weff2d = jax.lax.dot_general(gates, w2_ref[0], (((1,), (0,)), ((), ())),
                                 preferred_element_type=jnp.float32)
    weff = weff2d.reshape(TB, OC, OC)
    out = jax.lax.dot_general(h, weff, (((2,), (1,)), ((0,), (0,))),
                              preferred_element_type=jnp.float32)
    out = out + bias_c[:, None, :]
    cf_ref[:, 0] = jnp.swapaxes(out, 1, 2)

    @pl.when(jnp.logical_and(g == G - 1, bt == nbt - 1))
    def _():
        def cv(v):
            m = jnp.mean(v, axis=1, keepdims=True)
            var = jnp.sum((v - m) ** 2, axis=1, keepdims=True) / (NE - 1)
            return var / (m * m + 1e-10)
        loss = 0.01 * (jnp.sum(cv(imp_ref[...]), keepdims=True)
                       + jnp.sum(cv(load_ref[...]), keepdims=True))
        loss_ref[...] = loss


def _moe_call(xT4, gin, w1, b1, w2p, b2r, wg, interpret=False):
    b = xT4.shape[1]
    nbt = b // TB
    grid = (G, nbt)
    out_shapes = (
        jax.ShapeDtypeStruct((b, G, OC, LP), jnp.float32),
        jax.ShapeDtypeStruct((G, b, NE), jnp.float32),
        jax.ShapeDtypeStruct((1, 1), jnp.float32),
    )
    return pl.pallas_call(
        functools.partial(_moe_body, nbt=nbt),
        grid=grid,
        in_specs=[
            pl.BlockSpec((1, TB, LL, DIM), lambda g, t: (g, t, 0, 0)),
            pl.BlockSpec((1, TB, DIM * 5), lambda g, t: (g, t, 0)),
            pl.BlockSpec((1, 3 * DIM, OC), lambda g, t: (g, 0, 0)),
            pl.BlockSpec((1, 1, OC), lambda g, t: (g, 0, 0)),
            pl.BlockSpec((1, NE, OC * OC), lambda g, t: (g, 0, 0)),
            pl.BlockSpec((1, NE, OC), lambda g, t: (g, 0, 0)),
            pl.BlockSpec((1, DIM * 5, NE), lambda g, t: (g, 0, 0)),
        ],
        out_specs=(
            pl.BlockSpec((TB, 1, OC, LP), lambda g, t: (t, g, 0, 0)),
            pl.BlockSpec((1, TB, NE), lambda g, t: (g, t, 0)),
            pl.BlockSpec((1, 1), lambda g, t: (0, 0)),
        ),
        out_shape=out_shapes,
        scratch_shapes=[
            pltpu.VMEM((G, NE), jnp.float32),
            pltpu.VMEM((G, NE), jnp.float32),
        ],
        interpret=interpret,
    )(xT4, gin, w1, b1, w2p, b2r, wg)


def _run(x, conv1_w, conv1_b, conv2_w, conv2_b, w_gate, interpret=False):
    b = x.shape[0]
    xT4 = x.transpose(0, 2, 1).reshape(b, LL, G, DIM).transpose(2, 0, 1, 3)
    gin = x[:, :, LL - 6:LL - 1].reshape(b, G, DIM * 5).transpose(1, 0, 2)
    w1 = conv1_w.transpose(0, 3, 2, 1).reshape(G, 3 * DIM, OC)
    b1 = conv1_b.reshape(G, 1, OC)
    w2p = conv2_w[..., 0].reshape(G, OC, NE, OC).transpose(0, 2, 3, 1) \
        .reshape(G, NE, OC * OC)
    b2r = conv2_b.reshape(G, OC, NE).transpose(0, 2, 1)
    cf, gates_all, loss2d = _moe_call(xT4, gin, w1, b1, w2p, b2r,
                                      w_gate, interpret=interpret)
    return cf, loss2d[0, 0], gates_all.transpose(1, 2, 0)


def kernel(x, conv1_w, conv1_b, conv2_w, conv2_b, w_gate):
    return _run(x, conv1_w, conv1_b, conv2_w, conv2_b, w_gate)


# speedup vs baseline: 3.1635x; 1.3519x over previous
import functools

import jax
import jax.numpy as jnp
from jax.experimental import pallas as pl
from jax.experimental.pallas import tpu as pltpu

G = 4
DIM = 32
NE = 8
OC = 64
K = 4
LL = 128
LP = 126
TB = 32


def _moe_body(x_ref, w1_ref, b1_ref, w2_ref, b2_ref, wg_ref,
              cf_ref, gates_ref, loss_ref, imp_ref, load_ref, *, nbt):
    g = pl.program_id(0)
    bt = pl.program_id(1)
    xb = x_ref[:, 0]

    wg = wg_ref[0]
    logits = jnp.zeros((TB, NE), jnp.float32)
    for t in range(5):
        logits += jax.lax.dot_general(
            xb[:, :, LL - 6 + t], wg[:, t, :], (((1,), (0,)), ((), ())),
            preferred_element_type=jnp.float32)
    logits = logits - jnp.max(logits, axis=1, keepdims=True)
    ex = jnp.exp(logits)
    probs = ex / jnp.sum(ex, axis=1, keepdims=True)

    iota = jax.lax.broadcasted_iota(jnp.int32, (TB, NE), 1)
    vals = probs
    sel = jnp.zeros((TB, NE), jnp.bool_)
    for _ in range(K):
        m = jnp.max(vals, axis=1, keepdims=True)
        cand = jnp.where(vals == m, iota, NE)
        first = jnp.min(cand, axis=1, keepdims=True)
        pick = iota == first
        sel = jnp.logical_or(sel, pick)
        vals = jnp.where(pick, -jnp.inf, vals)
    tk = jnp.where(sel, probs, 0.0)
    denom = jnp.sum(tk, axis=1, keepdims=True) + 1e-6
    gates = tk / denom
    gates_ref[0] = gates

    imp_part = jnp.sum(gates, axis=0, keepdims=True)
    load_part = jnp.sum((gates > 0).astype(jnp.float32), axis=0,
                        keepdims=True)

    @pl.when(bt == 0)
    def _():
        imp_ref[pl.ds(g, 1), :] = imp_part
        load_ref[pl.ds(g, 1), :] = load_part

    @pl.when(bt != 0)
    def _():
        imp_ref[pl.ds(g, 1), :] = imp_ref[pl.ds(g, 1), :] + imp_part
        load_ref[pl.ds(g, 1), :] = load_ref[pl.ds(g, 1), :] + load_part

    w1 = w1_ref[0]
    hf = [jax.lax.dot_general(xb, w1[t], (((1,), (0,)), ((), ())),
                              preferred_element_type=jnp.float32)
          for t in range(3)]
    hpre = hf[0][:, 0:LP] + hf[1][:, 1:LP + 1] + hf[2][:, 2:LP + 2]
    h = jnp.tanh(hpre + b1_ref[0][None])

    bias_c = jax.lax.dot_general(gates, b2_ref[0], (((1,), (0,)), ((), ())),
                                 preferred_element_type=jnp.float32)
    weff2d = jax.lax.dot_general(gates, w2_ref[0], (((1,), (0,)), ((), ())),
                                 preferred_element_type=jnp.float32)
    weff = weff2d.reshape(TB, OC, OC)
    out = jax.lax.dot_general(h, weff, (((2,), (1,)), ((0,), (0,))),
                              preferred_element_type=jnp.float32)
    out = out + bias_c[:, None, :]
    cf_ref[:, 0] = jnp.swapaxes(out, 1, 2)

    @pl.when(jnp.logical_and(g == G - 1, bt == nbt - 1))
    def _():
        def cv(v):
            m = jnp.mean(v, axis=1, keepdims=True)
            var = jnp.sum((v - m) ** 2, axis=1, keepdims=True) / (NE - 1)
            return var / (m * m + 1e-10)
        loss = 0.01 * (jnp.sum(cv(imp_ref[...]), keepdims=True)
                       + jnp.sum(cv(load_ref[...]), keepdims=True))
        loss_ref[...] = loss


def _moe_call(x4, w1, b1, w2p, b2r, wg, interpret=False):
    b = x4.shape[0]
    nbt = b // TB
    grid = (G, nbt)
    out_shapes = (
        jax.ShapeDtypeStruct((b, G, OC, LP), jnp.float32),
        jax.ShapeDtypeStruct((G, b, NE), jnp.float32),
        jax.ShapeDtypeStruct((1, 1), jnp.float32),
    )
    return pl.pallas_call(
        functools.partial(_moe_body, nbt=nbt),
        grid=grid,
        in_specs=[
            pl.BlockSpec((TB, 1, DIM, LL), lambda g, t: (t, g, 0, 0)),
            pl.BlockSpec((1, 3, DIM, OC), lambda g, t: (g, 0, 0, 0)),
            pl.BlockSpec((1, 1, OC), lambda g, t: (g, 0, 0)),
            pl.BlockSpec((1, NE, OC * OC), lambda g, t: (g, 0, 0)),
            pl.BlockSpec((1, NE, OC), lambda g, t: (g, 0, 0)),
            pl.BlockSpec((1, DIM, 5, NE), lambda g, t: (g, 0, 0, 0)),
        ],
        out_specs=(
            pl.BlockSpec((TB, 1, OC, LP), lambda g, t: (t, g, 0, 0)),
            pl.BlockSpec((1, TB, NE), lambda g, t: (g, t, 0)),
            pl.BlockSpec((1, 1), lambda g, t: (0, 0)),
        ),
        out_shape=out_shapes,
        scratch_shapes=[
            pltpu.VMEM((G, NE), jnp.float32),
            pltpu.VMEM((G, NE), jnp.float32),
        ],
        interpret=interpret,
        compiler_params=pltpu.CompilerParams(
            fuse_transposed_lhs_in_matmul=True),
    )(x4, w1, b1, w2p, b2r, wg)


def _run(x, conv1_w, conv1_b, conv2_w, conv2_b, w_gate, interpret=False):
    b = x.shape[0]
    x4 = x.reshape(b, G, DIM, LL)
    w1 = conv1_w.transpose(0, 3, 2, 1)
    b1 = conv1_b.reshape(G, 1, OC)
    w2p = conv2_w[..., 0].reshape(G, OC, NE, OC).transpose(0, 2, 3, 1) \
        .reshape(G, NE, OC * OC)
    b2r = conv2_b.reshape(G, OC, NE).transpose(0, 2, 1)
    wg = w_gate.reshape(G, DIM, 5, NE)
    cf, gates_all, loss2d = _moe_call(x4, w1, b1, w2p, b2r,
                                      wg, interpret=interpret)
    return cf, loss2d[0, 0], gates_all.transpose(1, 2, 0)


def kernel(x, conv1_w, conv1_b, conv2_w, conv2_b, w_gate):
    return _run(x, conv1_w, conv1_b, conv2_w, conv2_b, w_gate)


# speedup vs baseline: 4.0669x; 1.2856x over previous
import functools

import jax
import jax.numpy as jnp
from jax import lax
from jax.experimental import pallas as pl
from jax.experimental.pallas import tpu as pltpu
from jax.experimental.pallas import tpu_sc as plsc

G = 4
DIM = 32
NE = 8
OC = 64
K = 4
LL = 128
LP = 126
TB = 32
SCB = 128
NW = 32
NBLK = 8


def _route_body(x_ref, wg_ref, gates_ref, part_ref, xw_v, wgv, gv, pacc, sem):
    wid = lax.axis_index("s") * 2 + lax.axis_index("c")
    g = wid // NBLK
    blk = wid % NBLK
    tok0 = blk * SCB

    pltpu.async_copy(
        x_ref.at[pl.ds(tok0, SCB), pl.ds(g * DIM * 8, DIM * 8)],
        xw_v, sem).wait()
    pltpu.async_copy(wg_ref.at[g], wgv, sem).wait()

    zero = jnp.zeros((16,), jnp.float32)
    for j in range(2):
        for e in range(NE):
            pacc[j, e] = zero

    lane = lax.broadcasted_iota(jnp.int32, (16,), 0)

    def chunk_body(c, _):
        idx_tok = c * 16 + lane

        def i_body(i, acc):
            for t in range(5):
                col = jnp.full((16,), i * 8 + t + 2, jnp.int32)
                xv = plsc.load_gather(xw_v, [idx_tok, col])
                base = i * 40 + t * NE
                for e in range(NE):
                    ws = plsc.load_gather(wgv, [jnp.full((16,), base + e,
                                                         jnp.int32)])
                    acc = tuple(
                        acc[q] + xv * ws if q == e else acc[q]
                        for q in range(NE))
            return acc

        le = lax.fori_loop(0, DIM, i_body,
                           tuple(zero for _ in range(NE)))

        m = le[0]
        for e in range(1, NE):
            m = jnp.maximum(m, le[e])
        z = [jnp.exp(le[e] - m) for e in range(NE)]
        s = z[0]
        for e in range(1, NE):
            s = s + z[e]
        p = [z[e] / s for e in range(NE)]

        work = list(p)
        sel = [jnp.zeros((16,), jnp.bool_) for _ in range(NE)]
        neg = jnp.full((16,), -1e30, jnp.float32)
        for _ in range(K):
            mx = work[0]
            for e in range(1, NE):
                mx = jnp.maximum(mx, work[e])
            found = jnp.zeros((16,), jnp.bool_)
            for e in range(NE):
                is_e = jnp.logical_and(work[e] == mx,
                                       jnp.logical_not(found))
                found = jnp.logical_or(found, is_e)
                sel[e] = jnp.logical_or(sel[e], is_e)
                work[e] = jnp.where(is_e, neg, work[e])

        tk = [jnp.where(sel[e], p[e], 0.0) for e in range(NE)]
        denom = tk[0]
        for e in range(1, NE):
            denom = denom + tk[e]
        denom = denom + 1e-6
        one = jnp.ones((16,), jnp.float32)
        for e in range(NE):
            gv_e = tk[e] / denom
            plsc.store_scatter(gv, [idx_tok, jnp.full((16,), e, jnp.int32)],
                               gv_e)
            plsc.addupdate(pacc.at[0, e], gv_e)
            plsc.addupdate(pacc.at[1, e],
                           jnp.where(gv_e > 0.0, one, 0.0))
        return ()

    lax.fori_loop(0, SCB // 16, chunk_body, ())

    pltpu.sync_copy(gv, gates_ref.at[g, pl.ds(tok0, SCB), :])
    pltpu.sync_copy(pacc, part_ref.at[:, g, :, blk, :])


def _route_call(xwin, wgr, interpret=False):
    b = xwin.shape[0]
    mesh = plsc.VectorSubcoreMesh(core_axis_name="c", subcore_axis_name="s",
                                  num_cores=2, num_subcores=16)
    return pl.kernel(
        _route_body,
        interpret=interpret,
        compiler_params=pltpu.CompilerParams(needs_layout_passes=False),
        out_type=(
            jax.ShapeDtypeStruct((G, b, NE), jnp.float32),
            jax.ShapeDtypeStruct((2, G, NE, NBLK, 16), jnp.float32),
        ),
        mesh=mesh,
        scratch_types=[
            pltpu.VMEM((SCB, DIM * 8), jnp.float32),
            pltpu.VMEM((DIM * 5 * NE,), jnp.float32),
            pltpu.VMEM((SCB, NE), jnp.float32),
            pltpu.VMEM((2, NE, 16), jnp.float32),
            pltpu.SemaphoreType.DMA,
        ],
    )(xwin, wgr)


def _moe_body(x_ref, w1_ref, b1_ref, w2_ref, b2_ref, gates_ref, part_ref,
              cf_ref, loss_ref, *, nbt):
    g = pl.program_id(0)
    bt = pl.program_id(1)
    xb = x_ref[:, 0]
    gates = gates_ref[0]

    w1 = w1_ref[0]
    hf = [jax.lax.dot_general(xb, w1[t], (((1,), (0,)), ((), ())),
                              preferred_element_type=jnp.float32)
          for t in range(3)]
    hpre = hf[0][:, 0:LP] + hf[1][:, 1:LP + 1] + hf[2][:, 2:LP + 2]
    h = jnp.tanh(hpre + b1_ref[0][None])

    bias_c = jax.lax.dot_general(gates, b2_ref[0], (((1,), (0,)), ((), ())),
                                 preferred_element_type=jnp.float32)
    weff2d = jax.lax.dot_general(gates, w2_ref[0], (((1,), (0,)), ((), ())),
                                 preferred_element_type=jnp.float32)
    weff = weff2d.reshape(TB, OC, OC)
    out = jax.lax.dot_general(h, weff, (((2,), (1,)), ((0,), (0,))),
                              preferred_element_type=jnp.float32)
    out = out + bias_c[:, None, :]
    cf_ref[:, 0] = jnp.swapaxes(out, 1, 2)

    @pl.when(jnp.logical_and(g == G - 1, bt == nbt - 1))
    def _():
        sums = jnp.sum(part_ref[...], axis=3)

        def cv(v):
            m = jnp.mean(v, axis=1, keepdims=True)
            var = jnp.sum((v - m) ** 2, axis=1, keepdims=True) / (NE - 1)
            return var / (m * m + 1e-10)
        loss = 0.01 * (jnp.sum(cv(sums[0]), keepdims=True)
                       + jnp.sum(cv(sums[1]), keepdims=True))
        loss_ref[...] = loss


def _moe_call(x4, w1, b1, w2p, b2r, gates_all, part, interpret=False):
    b = x4.shape[0]
    nbt = b // TB
    grid = (G, nbt)
    out_shapes = (
        jax.ShapeDtypeStruct((b, G, OC, LP), jnp.float32),
        jax.ShapeDtypeStruct((1, 1), jnp.float32),
    )
    return pl.pallas_call(
        functools.partial(_moe_body, nbt=nbt),
        grid=grid,
        in_specs=[
            pl.BlockSpec((TB, 1, DIM, LL), lambda g, t: (t, g, 0, 0)),
            pl.BlockSpec((1, 3, DIM, OC), lambda g, t: (g, 0, 0, 0)),
            pl.BlockSpec((1, 1, OC), lambda g, t: (g, 0, 0)),
            pl.BlockSpec((1, NE, OC * OC), lambda g, t: (g, 0, 0)),
            pl.BlockSpec((1, NE, OC), lambda g, t: (g, 0, 0)),
            pl.BlockSpec((1, TB, NE), lambda g, t: (g, t, 0)),
            pl.BlockSpec((2, G, NE, NBLK * 16), lambda g, t: (0, 0, 0, 0)),
        ],
        out_specs=(
            pl.BlockSpec((TB, 1, OC, LP), lambda g, t: (t, g, 0, 0)),
            pl.BlockSpec((1, 1), lambda g, t: (0, 0)),
        ),
        out_shape=out_shapes,
        interpret=interpret,
    )(x4, w1, b1, w2p, b2r, gates_all, part)


def _run(x, conv1_w, conv1_b, conv2_w, conv2_b, w_gate, interpret=False):
    b = x.shape[0]
    x4 = x.reshape(b, G, DIM, LL)
    w1 = conv1_w.transpose(0, 3, 2, 1)
    b1 = conv1_b.reshape(G, 1, OC)
    w2p = conv2_w[..., 0].reshape(G, OC, NE, OC).transpose(0, 2, 3, 1) \
        .reshape(G, NE, OC * OC)
    b2r = conv2_b.reshape(G, OC, NE).transpose(0, 2, 1)
    wgr = lax.reduce_precision(w_gate.reshape(G, DIM * 5 * NE), 8, 7)
    xwin = lax.reduce_precision(
        x4[:, :, :, LL - 8:].reshape(b, G * DIM * 8), 8, 7)
    gates_all, part = _route_call(xwin, wgr, interpret=interpret)
    cf, loss2d = _moe_call(x4, w1, b1, w2p, b2r, gates_all,
                           part.reshape(2, G, NE, NBLK * 16),
                           interpret=interpret)
    return cf, loss2d[0, 0], gates_all.transpose(1, 2, 0)


def kernel(x, conv1_w, conv1_b, conv2_w, conv2_b, w_gate):
    return _run(x, conv1_w, conv1_b, conv2_w, conv2_b, w_gate)


# speedup vs baseline: 4.2553x; 1.0463x over previous
import functools

import jax
import jax.numpy as jnp
from jax import lax
from jax.experimental import pallas as pl
from jax.experimental.pallas import tpu as pltpu
from jax.experimental.pallas import tpu_sc as plsc

G = 4
DIM = 32
NE = 8
OC = 64
K = 4
LL = 128
LP = 126
TB = 64
SCB = 128
NW = 32
NBLK = 8


def _route_body(x_ref, wg_ref, gates_ref, part_ref, xw_v, wgv, gv, pacc, sem):
    wid = lax.axis_index("s") * 2 + lax.axis_index("c")
    g = wid // NBLK
    blk = wid % NBLK
    tok0 = blk * SCB

    pltpu.async_copy(
        x_ref.at[pl.ds(tok0, SCB), pl.ds(g * DIM * 8, DIM * 8)],
        xw_v, sem).wait()
    pltpu.async_copy(wg_ref.at[g], wgv, sem).wait()

    zero = jnp.zeros((16,), jnp.float32)
    for j in range(2):
        for e in range(NE):
            pacc[j, e] = zero

    lane = lax.broadcasted_iota(jnp.int32, (16,), 0)

    def chunk_body(c, _):
        idx_tok = c * 16 + lane

        def i_body(i, acc):
            for t in range(5):
                col = jnp.full((16,), i * 8 + t + 2, jnp.int32)
                xv = plsc.load_gather(xw_v, [idx_tok, col])
                base = i * 40 + t * NE
                for e in range(NE):
                    ws = plsc.load_gather(wgv, [jnp.full((16,), base + e,
                                                         jnp.int32)])
                    acc = tuple(
                        acc[q] + xv * ws if q == e else acc[q]
                        for q in range(NE))
            return acc

        le = lax.fori_loop(0, DIM, i_body,
                           tuple(zero for _ in range(NE)))

        m = le[0]
        for e in range(1, NE):
            m = jnp.maximum(m, le[e])
        z = [jnp.exp(le[e] - m) for e in range(NE)]
        s = z[0]
        for e in range(1, NE):
            s = s + z[e]
        p = [z[e] / s for e in range(NE)]

        work = list(p)
        sel = [jnp.zeros((16,), jnp.bool_) for _ in range(NE)]
        neg = jnp.full((16,), -1e30, jnp.float32)
        for _ in range(K):
            mx = work[0]
            for e in range(1, NE):
                mx = jnp.maximum(mx, work[e])
            found = jnp.zeros((16,), jnp.bool_)
            for e in range(NE):
                is_e = jnp.logical_and(work[e] == mx,
                                       jnp.logical_not(found))
                found = jnp.logical_or(found, is_e)
                sel[e] = jnp.logical_or(sel[e], is_e)
                work[e] = jnp.where(is_e, neg, work[e])

        tk = [jnp.where(sel[e], p[e], 0.0) for e in range(NE)]
        denom = tk[0]
        for e in range(1, NE):
            denom = denom + tk[e]
        denom = denom + 1e-6
        one = jnp.ones((16,), jnp.float32)
        for e in range(NE):
            gv_e = tk[e] / denom
            plsc.store_scatter(gv, [idx_tok, jnp.full((16,), e, jnp.int32)],
                               gv_e)
            plsc.addupdate(pacc.at[0, e], gv_e)
            plsc.addupdate(pacc.at[1, e],
                           jnp.where(gv_e > 0.0, one, 0.0))
        return ()

    lax.fori_loop(0, SCB // 16, chunk_body, ())

    pltpu.sync_copy(gv, gates_ref.at[g, pl.ds(tok0, SCB), :])
    pltpu.sync_copy(pacc, part_ref.at[:, g, :, blk, :])


def _route_call(xwin, wgr, interpret=False):
    b = xwin.shape[0]
    mesh = plsc.VectorSubcoreMesh(core_axis_name="c", subcore_axis_name="s",
                                  num_cores=2, num_subcores=16)
    return pl.kernel(
        _route_body,
        interpret=interpret,
        compiler_params=pltpu.CompilerParams(needs_layout_passes=False),
        out_type=(
            jax.ShapeDtypeStruct((G, b, NE), jnp.float32),
            jax.ShapeDtypeStruct((2, G, NE, NBLK, 16), jnp.float32),
        ),
        mesh=mesh,
        scratch_types=[
            pltpu.VMEM((SCB, DIM * 8), jnp.float32),
            pltpu.VMEM((DIM * 5 * NE,), jnp.float32),
            pltpu.VMEM((SCB, NE), jnp.float32),
            pltpu.VMEM((2, NE, 16), jnp.float32),
            pltpu.SemaphoreType.DMA,
        ],
    )(xwin, wgr)


def _moe_body(x_ref, w1_ref, b1_ref, w2_ref, b2_ref, gates_ref, part_ref,
              cf_ref, loss_ref, *, nbt):
    g = pl.program_id(0)
    bt = pl.program_id(1)
    xb = x_ref[:, 0]
    gates = gates_ref[0]

    w1 = w1_ref[0]
    hf = [jax.lax.dot_general(xb, w1[t], (((1,), (0,)), ((), ())),
                              preferred_element_type=jnp.float32)
          for t in range(3)]
    hpre = hf[0][:, 0:LP] + hf[1][:, 1:LP + 1] + hf[2][:, 2:LP + 2]
    h = jnp.tanh(hpre + b1_ref[0][None])

    bias_c = jax.lax.dot_general(gates, b2_ref[0], (((1,), (0,)), ((), ())),
                                 preferred_element_type=jnp.float32)
    weff2d = jax.lax.dot_general(gates, w2_ref[0], (((1,), (0,)), ((), ())),
                                 preferred_element_type=jnp.float32)
    weff = weff2d.reshape(TB, OC, OC)
    out = jax.lax.dot_general(h, weff, (((2,), (1,)), ((0,), (0,))),
                              preferred_element_type=jnp.float32)
    out = out + bias_c[:, None, :]
    cf_ref[:, 0] = jnp.swapaxes(out, 1, 2)

    @pl.when(jnp.logical_and(g == G - 1, bt == nbt - 1))
    def _():
        sums = jnp.sum(part_ref[...], axis=3)

        def cv(v):
            m = jnp.mean(v, axis=1, keepdims=True)
            var = jnp.sum((v - m) ** 2, axis=1, keepdims=True) / (NE - 1)
            return var / (m * m + 1e-10)
        loss = 0.01 * (jnp.sum(cv(sums[0]), keepdims=True)
                       + jnp.sum(cv(sums[1]), keepdims=True))
        loss_ref[...] = loss


def _moe_call(x4, w1, b1, w2p, b2r, gates_all, part, interpret=False):
    b = x4.shape[0]
    nbt = b // TB
    grid = (G, nbt)
    out_shapes = (
        jax.ShapeDtypeStruct((b, G, OC, LP), jnp.float32),
        jax.ShapeDtypeStruct((1, 1), jnp.float32),
    )
    return pl.pallas_call(
        functools.partial(_moe_body, nbt=nbt),
        grid=grid,
        in_specs=[
            pl.BlockSpec((TB, 1, DIM, LL), lambda g, t: (t, g, 0, 0)),
            pl.BlockSpec((1, 3, DIM, OC), lambda g, t: (g, 0, 0, 0)),
            pl.BlockSpec((1, 1, OC), lambda g, t: (g, 0, 0)),
            pl.BlockSpec((1, NE, OC * OC), lambda g, t: (g, 0, 0)),
            pl.BlockSpec((1, NE, OC), lambda g, t: (g, 0, 0)),
            pl.BlockSpec((1, TB, NE), lambda g, t: (g, t, 0)),
            pl.BlockSpec((2, G, NE, NBLK * 16), lambda g, t: (0, 0, 0, 0)),
        ],
        out_specs=(
            pl.BlockSpec((TB, 1, OC, LP), lambda g, t: (t, g, 0, 0)),
            pl.BlockSpec((1, 1), lambda g, t: (0, 0)),
        ),
        out_shape=out_shapes,
        interpret=interpret,
    )(x4, w1, b1, w2p, b2r, gates_all, part)


def _run(x, conv1_w, conv1_b, conv2_w, conv2_b, w_gate, interpret=False):
    b = x.shape[0]
    x4 = x.reshape(b, G, DIM, LL)
    w1 = conv1_w.transpose(0, 3, 2, 1)
    b1 = conv1_b.reshape(G, 1, OC)
    w2p = conv2_w[..., 0].reshape(G, OC, NE, OC).transpose(0, 2, 3, 1) \
        .reshape(G, NE, OC * OC)
    b2r = conv2_b.reshape(G, OC, NE).transpose(0, 2, 1)
    wgr = lax.reduce_precision(w_gate.reshape(G, DIM * 5 * NE), 8, 7)
    xwin = lax.reduce_precision(
        x4[:, :, :, LL - 8:].reshape(b, G * DIM * 8), 8, 7)
    gates_all, part = _route_call(xwin, wgr, interpret=interpret)
    cf, loss2d = _moe_call(x4, w1, b1, w2p, b2r, gates_all,
                           part.reshape(2, G, NE, NBLK * 16),
                           interpret=interpret)
    return cf, loss2d[0, 0], gates_all.transpose(1, 2, 0)


def kernel(x, conv1_w, conv1_b, conv2_w, conv2_b, w_gate):
    return _run(x, conv1_w, conv1_b, conv2_w, conv2_b, w_gate)


# speedup vs baseline: 4.4319x; 1.0415x over previous
import functools

import jax
import jax.numpy as jnp
from jax import lax
from jax.experimental import pallas as pl
from jax.experimental.pallas import tpu as pltpu
from jax.experimental.pallas import tpu_sc as plsc

G = 4
DIM = 32
NE = 8
OC = 64
K = 4
LL = 128
LP = 126
TB = 64
SCB = 128
NW = 32
NBLK = 8


def _route_body(x_ref, wg_ref, gates_ref, part_ref, xw_v, wgv, gv, pacc, sem):
    wid = lax.axis_index("s") * 2 + lax.axis_index("c")
    g = wid // NBLK
    blk = wid % NBLK
    tok0 = blk * SCB

    pltpu.async_copy(
        x_ref.at[pl.ds(tok0, SCB), pl.ds(g * DIM * 8, DIM * 8)],
        xw_v, sem).wait()
    pltpu.async_copy(wg_ref.at[g], wgv, sem).wait()

    zero = jnp.zeros((16,), jnp.float32)
    for j in range(2):
        for e in range(NE):
            pacc[j, e] = zero

    lane = lax.broadcasted_iota(jnp.int32, (16,), 0)

    def chunk_body(c, _):
        idx_tok = c * 16 + lane

        def i_body(i, acc):
            for t in range(5):
                col = jnp.full((16,), i * 8 + t + 2, jnp.int32)
                xv = plsc.load_gather(xw_v, [idx_tok, col])
                base = i * 40 + t * NE
                for e in range(NE):
                    ws = plsc.load_gather(wgv, [jnp.full((16,), base + e,
                                                         jnp.int32)])
                    acc = tuple(
                        acc[q] + xv * ws if q == e else acc[q]
                        for q in range(NE))
            return acc

        le = lax.fori_loop(0, DIM, i_body,
                           tuple(zero for _ in range(NE)))

        m = le[0]
        for e in range(1, NE):
            m = jnp.maximum(m, le[e])
        z = [jnp.exp(le[e] - m) for e in range(NE)]
        s = z[0]
        for e in range(1, NE):
            s = s + z[e]
        p = [z[e] / s for e in range(NE)]

        work = list(p)
        sel = [jnp.zeros((16,), jnp.bool_) for _ in range(NE)]
        neg = jnp.full((16,), -1e30, jnp.float32)
        for _ in range(K):
            mx = work[0]
            for e in range(1, NE):
                mx = jnp.maximum(mx, work[e])
            found = jnp.zeros((16,), jnp.bool_)
            for e in range(NE):
                is_e = jnp.logical_and(work[e] == mx,
                                       jnp.logical_not(found))
                found = jnp.logical_or(found, is_e)
                sel[e] = jnp.logical_or(sel[e], is_e)
                work[e] = jnp.where(is_e, neg, work[e])

        tk = [jnp.where(sel[e], p[e], 0.0) for e in range(NE)]
        denom = tk[0]
        for e in range(1, NE):
            denom = denom + tk[e]
        denom = denom + 1e-6
        one = jnp.ones((16,), jnp.float32)
        for e in range(NE):
            gv_e = tk[e] / denom
            plsc.store_scatter(gv, [idx_tok, jnp.full((16,), e, jnp.int32)],
                               gv_e)
            plsc.addupdate(pacc.at[0, e], gv_e)
            plsc.addupdate(pacc.at[1, e],
                           jnp.where(gv_e > 0.0, one, 0.0))
        return ()

    lax.fori_loop(0, SCB // 16, chunk_body, ())

    pltpu.sync_copy(gv, gates_ref.at[g, pl.ds(tok0, SCB), :])
    pltpu.sync_copy(pacc, part_ref.at[:, g, :, blk, :])


def _route_call(xwin, wgr, interpret=False):
    b = xwin.shape[0]
    mesh = plsc.VectorSubcoreMesh(core_axis_name="c", subcore_axis_name="s",
                                  num_cores=2, num_subcores=16)
    return pl.kernel(
        _route_body,
        interpret=interpret,
        compiler_params=pltpu.CompilerParams(needs_layout_passes=False),
        out_type=(
            jax.ShapeDtypeStruct((G, b, NE), jnp.float32),
            jax.ShapeDtypeStruct((2, G, NE, NBLK, 16), jnp.float32),
        ),
        mesh=mesh,
        scratch_types=[
            pltpu.VMEM((SCB, DIM * 8), jnp.float32),
            pltpu.VMEM((DIM * 5 * NE,), jnp.float32),
            pltpu.VMEM((SCB, NE), jnp.float32),
            pltpu.VMEM((2, NE, 16), jnp.float32),
            pltpu.SemaphoreType.DMA,
        ],
    )(xwin, wgr)


def _moe_body(x_ref, w1_ref, b1_ref, w2_ref, b2_ref, gates_ref, part_ref,
              cf_ref, loss_ref, *, nbt):
    g = pl.program_id(0)
    bt = pl.program_id(1)
    xb = x_ref[:, 0]
    gates = gates_ref[0]

    w1 = w1_ref[0]
    xb16 = xb.astype(jnp.bfloat16)
    hf = [jax.lax.dot_general(xb16, w1[t], (((1,), (0,)), ((), ())),
                              preferred_element_type=jnp.float32)
          for t in range(3)]
    hpre = hf[0][:, 0:LP] + hf[1][:, 1:LP + 1] + hf[2][:, 2:LP + 2]
    h = jnp.tanh(hpre + b1_ref[0][None])

    bias_c = jax.lax.dot_general(gates, b2_ref[0], (((1,), (0,)), ((), ())),
                                 preferred_element_type=jnp.float32)
    weff2d = jax.lax.dot_general(gates.astype(jnp.bfloat16), w2_ref[0],
                                 (((1,), (0,)), ((), ())),
                                 preferred_element_type=jnp.float32)
    weff = weff2d.astype(jnp.bfloat16).reshape(TB, OC, OC)
    out = jax.lax.dot_general(h.astype(jnp.bfloat16), weff,
                              (((2,), (1,)), ((0,), (0,))),
                              preferred_element_type=jnp.float32)
    out = out + bias_c[:, None, :]
    cf_ref[:, 0] = jnp.swapaxes(out, 1, 2)

    @pl.when(jnp.logical_and(g == G - 1, bt == nbt - 1))
    def _():
        sums = jnp.sum(part_ref[...], axis=3)

        def cv(v):
            m = jnp.mean(v, axis=1, keepdims=True)
            var = jnp.sum((v - m) ** 2, axis=1, keepdims=True) / (NE - 1)
            return var / (m * m + 1e-10)
        loss = 0.01 * (jnp.sum(cv(sums[0]), keepdims=True)
                       + jnp.sum(cv(sums[1]), keepdims=True))
        loss_ref[...] = loss


def _moe_call(x4, w1, b1, w2p, b2r, gates_all, part, interpret=False):
    b = x4.shape[0]
    nbt = b // TB
    grid = (G, nbt)
    out_shapes = (
        jax.ShapeDtypeStruct((b, G, OC, LP), jnp.float32),
        jax.ShapeDtypeStruct((1, 1), jnp.float32),
    )
    return pl.pallas_call(
        functools.partial(_moe_body, nbt=nbt),
        grid=grid,
        in_specs=[
            pl.BlockSpec((TB, 1, DIM, LL), lambda g, t: (t, g, 0, 0)),
            pl.BlockSpec((1, 3, DIM, OC), lambda g, t: (g, 0, 0, 0)),
            pl.BlockSpec((1, 1, OC), lambda g, t: (g, 0, 0)),
            pl.BlockSpec((1, NE, OC * OC), lambda g, t: (g, 0, 0)),
            pl.BlockSpec((1, NE, OC), lambda g, t: (g, 0, 0)),
            pl.BlockSpec((1, TB, NE), lambda g, t: (g, t, 0)),
            pl.BlockSpec((2, G, NE, NBLK * 16), lambda g, t: (0, 0, 0, 0)),
        ],
        out_specs=(
            pl.BlockSpec((TB, 1, OC, LP), lambda g, t: (t, g, 0, 0)),
            pl.BlockSpec((1, 1), lambda g, t: (0, 0)),
        ),
        out_shape=out_shapes,
        interpret=interpret,
    )(x4, w1, b1, w2p, b2r, gates_all, part)


def _run(x, conv1_w, conv1_b, conv2_w, conv2_b, w_gate, interpret=False):
    b = x.shape[0]
    x4 = x.reshape(b, G, DIM, LL)
    w1 = conv1_w.transpose(0, 3, 2, 1).astype(jnp.bfloat16)
    b1 = conv1_b.reshape(G, 1, OC)
    w2p = conv2_w[..., 0].reshape(G, OC, NE, OC).transpose(0, 2, 3, 1) \
        .reshape(G, NE, OC * OC).astype(jnp.bfloat16)
    b2r = conv2_b.reshape(G, OC, NE).transpose(0, 2, 1)
    wgr = lax.reduce_precision(w_gate.reshape(G, DIM * 5 * NE), 8, 7)
    xwin = lax.reduce_precision(
        x4[:, :, :, LL - 8:].reshape(b, G * DIM * 8), 8, 7)
    gates_all, part = _route_call(xwin, wgr, interpret=interpret)
    cf, loss2d = _moe_call(x4, w1, b1, w2p, b2r, gates_all,
                           part.reshape(2, G, NE, NBLK * 16),
                           interpret=interpret)
    return cf, loss2d[0, 0], gates_all.transpose(1, 2, 0)


def kernel(x, conv1_w, conv1_b, conv2_w, conv2_b, w_gate):
    return _run(x, conv1_w, conv1_b, conv2_w, conv2_b, w_gate)
